# baseline (speedup 1.0000x reference)
import functools

import jax
import jax.numpy as jnp
from jax import lax
from jax.experimental import pallas as pl
from jax.experimental.pallas import tpu as pltpu
from jax.experimental.pallas import tpu_sc as plsc

BATCH = 16384
NT = 50
DIM = 32
NUM_CORES = 2
NUM_SUBCORES = 16
NW = NUM_CORES * NUM_SUBCORES
LANES = 128
BT_TILES = BATCH // LANES
BT_PER_W = BT_TILES // NW
UNITS = BT_PER_W * NT
NBUF = 8

_mesh = plsc.VectorSubcoreMesh(
    core_axis_name="c", subcore_axis_name="s",
    num_cores=NUM_CORES, num_subcores=NUM_SUBCORES)


@functools.partial(
    pl.kernel,
    out_type=jax.ShapeDtypeStruct((NT, DIM // 8, BT_TILES, 8, LANES),
                                  jnp.float32),
    mesh=_mesh,
    compiler_params=pltpu.CompilerParams(use_tc_tiling_on_sc=False,
                                         needs_layout_passes=False),
    scratch_types=[
        pltpu.VMEM((BT_PER_W, NT, LANES), jnp.int32),
        pltpu.VMEM((NBUF, LANES, DIM), jnp.float32),
        pltpu.VMEM((2, DIM // 8, 8, LANES), jnp.float32),
        pltpu.SemaphoreType.DMA,
        pltpu.SemaphoreType.DMA,
        pltpu.SemaphoreType.DMA,
    ],
)
def _gather_kernel(textT_hbm, table_hbm, out_hbm, idx_v, rows_v, tbuf_v,
                   gsem, wsem0, wsem1):
    wid = lax.axis_index("s") * NUM_CORES + lax.axis_index("c")
    iota = lax.iota(jnp.int32, 16)
    lane_ids = [iota + 16 * l0 for l0 in range(8)]

    for bti in range(BT_PER_W):
        bt = wid * BT_PER_W + bti
        pltpu.sync_copy(textT_hbm.at[:, pl.ds(bt * LANES, LANES)],
                        idx_v.at[bti])

    def fire_gather(u):
        t = u % NT
        bti = u // NT
        pltpu.async_copy(table_hbm.at[idx_v.at[bti, t]],
                         rows_v.at[u % NBUF], gsem)

    def wait_gather(u):
        t = u % NT
        bti = u // NT
        pltpu.make_async_copy(table_hbm.at[idx_v.at[bti, t]],
                              rows_v.at[u % NBUF], gsem).wait()

    def out_slot(u):
        t = u % NT
        bt = wid * BT_PER_W + u // NT
        return out_hbm.at[t, :, bt]

    for u in range(NBUF - 1):
        fire_gather(u)

    @pl.loop(0, UNITS)
    def _unit(u):
        @pl.when(u + NBUF - 1 < UNITS)
        def _():
            fire_gather(u + NBUF - 1)
        wait_gather(u)
        rows = rows_v.at[u % NBUF]
        tbuf = tbuf_v.at[u % 2]

        @pl.when((u >= 2) & (u % 2 == 0))
        def _():
            pltpu.make_async_copy(tbuf_v.at[0], out_slot(u), wsem0).wait()

        @pl.when((u >= 2) & (u % 2 == 1))
        def _():
            pltpu.make_async_copy(tbuf_v.at[1], out_slot(u), wsem1).wait()

        @plsc.parallel_loop(0, DIM, unroll=8)
        def _transpose(r):
            col = jnp.full((16,), r, jnp.int32)
            rt = r // 8
            s = r % 8
            for l0 in range(8):
                v = plsc.load_gather(rows, [lane_ids[l0], col])
                tbuf[rt, s, pl.ds(16 * l0, 16)] = v

        @pl.when(u % 2 == 0)
        def _():
            pltpu.async_copy(tbuf_v.at[0], out_slot(u), wsem0)

        @pl.when(u % 2 == 1)
        def _():
            pltpu.async_copy(tbuf_v.at[1], out_slot(u), wsem1)

    pltpu.make_async_copy(tbuf_v.at[0], out_hbm.at[0, :, 0], wsem0).wait()
    pltpu.make_async_copy(tbuf_v.at[1], out_hbm.at[0, :, 0], wsem1).wait()


def kernel(text, table):
    textT = text.T
    out5 = _gather_kernel(textT, table)
    return out5.transpose(2, 4, 0, 1, 3).reshape(BATCH, NT, DIM)


# speedup vs baseline: 1.0972x; 1.0972x over previous
import functools

import jax
import jax.numpy as jnp
from jax import lax
from jax.experimental import pallas as pl
from jax.experimental.pallas import tpu as pltpu
from jax.experimental.pallas import tpu_sc as plsc

BATCH = 16384
NT = 50
DIM = 32
VOCAB = 1000001
VPAD = 1000064
NTILES = VPAD // 128
NUM_CORES = 2
NUM_SUBCORES = 16
NW = NUM_CORES * NUM_SUBCORES
LANES = 128
BT_TILES = BATCH // LANES
BT_PER_W = BT_TILES // NW
UNITS = BT_PER_W * NT
NBUF = 4

_mesh = plsc.VectorSubcoreMesh(
    core_axis_name="c", subcore_axis_name="s",
    num_cores=NUM_CORES, num_subcores=NUM_SUBCORES)


@functools.partial(
    pl.kernel,
    out_type=jax.ShapeDtypeStruct((VPAD * DIM,), jnp.float32),
    mesh=_mesh,
    compiler_params=pltpu.CompilerParams(needs_layout_passes=False),
    scratch_types=[
        pltpu.VMEM((NBUF, DIM, 128), jnp.float32),
        pltpu.VMEM((128 * DIM,), jnp.float32),
        pltpu.VMEM((128 * DIM,), jnp.float32),
        pltpu.SemaphoreType.DMA,
        pltpu.SemaphoreType.DMA,
        pltpu.SemaphoreType.DMA,
    ],
)
def _tr_kernel(tableT_hbm, out_hbm, tiles_v, obuf0, obuf1, rsem, wsem0, wsem1):
    wid = lax.axis_index("s") * NUM_CORES + lax.axis_index("c")
    ntiles = jnp.where(wid < NTILES % NW, NTILES // NW + 1, NTILES // NW)
    start = wid * (NTILES // NW) + jnp.minimum(wid, NTILES % NW)
    iota = lax.iota(jnp.int32, 16)
    pos0 = iota * DIM

    def fire_read(j):
        pltpu.async_copy(tableT_hbm.at[:, pl.ds((start + j) * 128, 128)],
                         tiles_v.at[j % NBUF], rsem)

    def wait_read(j):
        pltpu.make_async_copy(tableT_hbm.at[:, pl.ds((start + j) * 128, 128)],
                              tiles_v.at[j % NBUF], rsem).wait()

    for j in range(NBUF - 1):
        @pl.when(j < ntiles)
        def _():
            fire_read(j)

    @pl.loop(0, NTILES // NW + 1)
    def _tile(j):
        @pl.when(j < ntiles)
        def _():
            @pl.when(j + NBUF - 1 < ntiles)
            def _():
                fire_read(j + NBUF - 1)
            wait_read(j)
            tile = tiles_v.at[j % NBUF]

            def do_half(obuf, wsem):
                @pl.when(j >= 2)
                def _():
                    pltpu.make_async_copy(
                        obuf, out_hbm.at[pl.ds(0, 128 * DIM)], wsem).wait()

                @plsc.parallel_loop(0, DIM, unroll=4)
                def _transpose(c):
                    for k in range(8):
                        v = tile[c, pl.ds(16 * k, 16)]
                        plsc.store_scatter(
                            obuf, [pos0 + (16 * k * DIM + c)], v)

                pltpu.async_copy(
                    obuf,
                    out_hbm.at[pl.ds((start + j) * 128 * DIM, 128 * DIM)],
                    wsem)

            @pl.when(j % 2 == 0)
            def _():
                do_half(obuf0, wsem0)

            @pl.when(j % 2 == 1)
            def _():
                do_half(obuf1, wsem1)

    @pl.when(ntiles >= 1)
    def _():
        pltpu.make_async_copy(obuf0, out_hbm.at[pl.ds(0, 128 * DIM)],
                              wsem0).wait()

    @pl.when(ntiles >= 2)
    def _():
        pltpu.make_async_copy(obuf1, out_hbm.at[pl.ds(0, 128 * DIM)],
                              wsem1).wait()


@functools.partial(
    pl.kernel,
    out_type=jax.ShapeDtypeStruct((NT, DIM // 8, BT_TILES, 8, LANES),
                                  jnp.float32),
    mesh=_mesh,
    compiler_params=pltpu.CompilerParams(use_tc_tiling_on_sc=False,
                                         needs_layout_passes=False),
    scratch_types=[
        pltpu.VMEM((BT_PER_W, NT, LANES), jnp.int32),
        pltpu.VMEM((NBUF, LANES, DIM), jnp.float32),
        pltpu.VMEM((2, DIM // 8, 8, LANES), jnp.float32),
        pltpu.SemaphoreType.DMA,
        pltpu.SemaphoreType.DMA,
        pltpu.SemaphoreType.DMA,
    ],
)
def _gather_kernel(textT_hbm, table_hbm, out_hbm, idx_v, rows_v, tbuf_v,
                   gsem, wsem0, wsem1):
    wid = lax.axis_index("s") * NUM_CORES + lax.axis_index("c")
    iota = lax.iota(jnp.int32, 16)
    lane_ids = [iota + 16 * l0 for l0 in range(8)]

    for bti in range(BT_PER_W):
        bt = wid * BT_PER_W + bti
        pltpu.sync_copy(textT_hbm.at[:, pl.ds(bt * LANES, LANES)],
                        idx_v.at[bti])

    def fire_gather(u):
        t = u % NT
        bti = u // NT
        pltpu.async_copy(table_hbm.at[idx_v.at[bti, t]],
                         rows_v.at[u % NBUF], gsem)

    def wait_gather(u):
        t = u % NT
        bti = u // NT
        pltpu.make_async_copy(table_hbm.at[idx_v.at[bti, t]],
                              rows_v.at[u % NBUF], gsem).wait()

    def out_slot(u):
        t = u % NT
        bt = wid * BT_PER_W + u // NT
        return out_hbm.at[t, :, bt]

    for u in range(NBUF - 1):
        fire_gather(u)

    @pl.loop(0, UNITS)
    def _unit(u):
        @pl.when(u + NBUF - 1 < UNITS)
        def _():
            fire_gather(u + NBUF - 1)
        wait_gather(u)
        rows = rows_v.at[u % NBUF]
        tbuf = tbuf_v.at[u % 2]

        @pl.when((u >= 2) & (u % 2 == 0))
        def _():
            pltpu.make_async_copy(tbuf_v.at[0], out_slot(u), wsem0).wait()

        @pl.when((u >= 2) & (u % 2 == 1))
        def _():
            pltpu.make_async_copy(tbuf_v.at[1], out_slot(u), wsem1).wait()

        @plsc.parallel_loop(0, DIM, unroll=4)
        def _transpose(r):
            col = jnp.full((16,), r, jnp.int32)
            rt = r // 8
            s = r % 8
            for l0 in range(8):
                v = plsc.load_gather(rows, [lane_ids[l0], col])
                tbuf[rt, s, pl.ds(16 * l0, 16)] = v

        @pl.when(u % 2 == 0)
        def _():
            pltpu.async_copy(tbuf_v.at[0], out_slot(u), wsem0)

        @pl.when(u % 2 == 1)
        def _():
            pltpu.async_copy(tbuf_v.at[1], out_slot(u), wsem1)

    pltpu.make_async_copy(tbuf_v.at[0], out_hbm.at[0, :, 0], wsem0).wait()
    pltpu.make_async_copy(tbuf_v.at[1], out_hbm.at[0, :, 0], wsem1).wait()


def kernel(text, table):
    textT = text.T
    tableT = jnp.swapaxes(table, 0, 1)
    table_rm = _tr_kernel(tableT).reshape(VPAD, DIM)
    out5 = _gather_kernel(textT, table_rm)
    return out5.transpose(2, 4, 0, 1, 3).reshape(BATCH, NT, DIM)


# speedup vs baseline: 4.3376x; 3.9534x over previous
import functools

import jax
import jax.numpy as jnp
from jax import lax
from jax.experimental import pallas as pl
from jax.experimental.pallas import tpu as pltpu
from jax.experimental.pallas import tpu_sc as plsc

BATCH = 16384
NT = 50
DIM = 32
VOCAB = 1000001
VPAD = 1000064
NTILES = VPAD // 128
NUM_CORES = 2
NUM_SUBCORES = 16
NW = NUM_CORES * NUM_SUBCORES
LANES = 128
BT_TILES = BATCH // LANES
BT_PER_W = BT_TILES // NW
UNITS = BT_PER_W * NT
NBUF = 4

_mesh = plsc.VectorSubcoreMesh(
    core_axis_name="c", subcore_axis_name="s",
    num_cores=NUM_CORES, num_subcores=NUM_SUBCORES)


@functools.partial(
    pl.kernel,
    out_type=jax.ShapeDtypeStruct((VPAD * DIM,), jnp.float32),
    mesh=_mesh,
    compiler_params=pltpu.CompilerParams(needs_layout_passes=False),
    scratch_types=[
        pltpu.VMEM((NBUF, DIM, 128), jnp.float32),
        pltpu.VMEM((128 * DIM,), jnp.float32),
        pltpu.VMEM((128 * DIM,), jnp.float32),
        pltpu.SemaphoreType.DMA,
        pltpu.SemaphoreType.DMA,
        pltpu.SemaphoreType.DMA,
    ],
)
def _tr_kernel(tableT_hbm, out_hbm, tiles_v, obuf0, obuf1, rsem, wsem0, wsem1):
    wid = lax.axis_index("s") * NUM_CORES + lax.axis_index("c")
    ntiles = jnp.where(wid < NTILES % NW, NTILES // NW + 1, NTILES // NW)
    start = wid * (NTILES // NW) + jnp.minimum(wid, NTILES % NW)
    iota = lax.iota(jnp.int32, 16)
    pos_k = [(iota + 16 * k) * DIM for k in range(8)]

    def fire_read(j):
        pltpu.async_copy(tableT_hbm.at[:, pl.ds((start + j) * 128, 128)],
                         tiles_v.at[j % NBUF], rsem)

    def wait_read(j):
        pltpu.make_async_copy(tableT_hbm.at[:, pl.ds((start + j) * 128, 128)],
                              tiles_v.at[j % NBUF], rsem).wait()

    for j in range(NBUF - 1):
        @pl.when(j < ntiles)
        def _():
            fire_read(j)

    @pl.loop(0, NTILES // NW + 1)
    def _tile(j):
        @pl.when(j < ntiles)
        def _():
            @pl.when(j + NBUF - 1 < ntiles)
            def _():
                fire_read(j + NBUF - 1)
            wait_read(j)
            tile = tiles_v.at[j % NBUF]

            def do_half(obuf, wsem):
                @pl.when(j >= 2)
                def _():
                    pltpu.make_async_copy(
                        obuf, out_hbm.at[pl.ds(0, 128 * DIM)], wsem).wait()

                @plsc.parallel_loop(0, DIM, unroll=4)
                def _transpose(c0):
                    cvec = (c0 + iota) & (DIM - 1)
                    for k in range(8):
                        v = plsc.load_gather(tile, [cvec, iota + 16 * k])
                        plsc.store_scatter(obuf, [pos_k[k] + cvec], v)

                pltpu.async_copy(
                    obuf,
                    out_hbm.at[pl.ds((start + j) * 128 * DIM, 128 * DIM)],
                    wsem)

            @pl.when(j % 2 == 0)
            def _():
                do_half(obuf0, wsem0)

            @pl.when(j % 2 == 1)
            def _():
                do_half(obuf1, wsem1)

    @pl.when(ntiles >= 1)
    def _():
        pltpu.make_async_copy(obuf0, out_hbm.at[pl.ds(0, 128 * DIM)],
                              wsem0).wait()

    @pl.when(ntiles >= 2)
    def _():
        pltpu.make_async_copy(obuf1, out_hbm.at[pl.ds(0, 128 * DIM)],
                              wsem1).wait()


@functools.partial(
    pl.kernel,
    out_type=jax.ShapeDtypeStruct((NT, DIM // 8, BT_TILES, 8, LANES),
                                  jnp.float32),
    mesh=_mesh,
    compiler_params=pltpu.CompilerParams(use_tc_tiling_on_sc=False,
                                         needs_layout_passes=False),
    scratch_types=[
        pltpu.VMEM((BT_PER_W, NT, LANES), jnp.int32),
        pltpu.VMEM((NBUF, LANES, DIM), jnp.float32),
        pltpu.VMEM((DIM // 8, 8, LANES), jnp.float32),
        pltpu.VMEM((DIM // 8, 8, LANES), jnp.float32),
        pltpu.SemaphoreType.DMA,
        pltpu.SemaphoreType.DMA,
        pltpu.SemaphoreType.DMA,
    ],
)
def _gather_kernel(textT_hbm, table_hbm, out_hbm, idx_v, rows_v, tbuf0, tbuf1,
                   gsem, wsem0, wsem1):
    wid = lax.axis_index("s") * NUM_CORES + lax.axis_index("c")
    iota = lax.iota(jnp.int32, 16)
    lane_ids = [iota + 16 * l0 for l0 in range(8)]

    for bti in range(BT_PER_W):
        bt = wid * BT_PER_W + bti
        pltpu.sync_copy(textT_hbm.at[:, pl.ds(bt * LANES, LANES)],
                        idx_v.at[bti])

    def fire_gather(u):
        t = u % NT
        bti = u // NT
        pltpu.async_copy(table_hbm.at[idx_v.at[bti, t]],
                         rows_v.at[u % NBUF], gsem)

    def wait_gather(u):
        t = u % NT
        bti = u // NT
        pltpu.make_async_copy(table_hbm.at[idx_v.at[bti, t]],
                              rows_v.at[u % NBUF], gsem).wait()

    def out_slot(u):
        t = u % NT
        bt = wid * BT_PER_W + u // NT
        return out_hbm.at[t, :, bt]

    for u in range(NBUF - 1):
        fire_gather(u)

    @pl.loop(0, UNITS)
    def _unit(u):
        @pl.when(u + NBUF - 1 < UNITS)
        def _():
            fire_gather(u + NBUF - 1)
        wait_gather(u)
        rows = rows_v.at[u % NBUF]

        def do_half(tbuf, wsem):
            @pl.when(u >= 2)
            def _():
                pltpu.make_async_copy(tbuf, out_slot(u), wsem).wait()

            @plsc.parallel_loop(0, DIM, unroll=4)
            def _transpose(c0):
                rvec = (c0 + iota) & (DIM - 1)
                rt = lax.shift_right_logical(rvec, 3)
                s = rvec & 7
                for q in range(8):
                    v = plsc.load_gather(rows, [lane_ids[q], rvec])
                    plsc.store_scatter(tbuf, [rt, s, lane_ids[q]], v)

            pltpu.async_copy(tbuf, out_slot(u), wsem)

        @pl.when(u % 2 == 0)
        def _():
            do_half(tbuf0, wsem0)

        @pl.when(u % 2 == 1)
        def _():
            do_half(tbuf1, wsem1)

    pltpu.make_async_copy(tbuf0, out_hbm.at[0, :, 0], wsem0).wait()
    pltpu.make_async_copy(tbuf1, out_hbm.at[0, :, 0], wsem1).wait()


def kernel(text, table):
    textT = text.T
    tableT = jnp.swapaxes(table, 0, 1)
    table_rm = _tr_kernel(tableT).reshape(VPAD, DIM)
    out5 = _gather_kernel(textT, table_rm)
    return out5.transpose(2, 4, 0, 1, 3).reshape(BATCH, NT, DIM)
